# baseline (speedup 1.0000x reference)
import math

import jax
import jax.numpy as jnp
from jax.experimental import pallas as pl
from jax.experimental.pallas import tpu as pltpu

K = 2
N = 10000
NFEAT = 128
NH = 64
NCLASS = 40
NLAYERS = 2
LAMDA = 0.5
ALPHA = 0.1

BM = 400
NT = N // BM
T0 = math.log(LAMDA / 1.0 + 1.0)
T1 = math.log(LAMDA / 2.0 + 1.0)


_PW = 8
_NP = NH // _PW


def _chol_solve(W, rows, lanes, eye):
    A = jax.lax.dot_general(W, W, (((0,), (0,)), ((), ())),
                            preferred_element_type=jnp.float32)
    A = A + 1e-4 * eye
    one = jnp.float32(1.0)
    zero = jnp.float32(0.0)
    lanes8 = jax.lax.broadcasted_iota(jnp.int32, (1, _PW), 1)
    rows8 = jax.lax.broadcasted_iota(jnp.int32, (_PW, 1), 0)
    recips = []
    lpans = []
    for p in range(_NP):
        c0 = p * _PW
        Apan = jax.lax.slice(A, (0, c0), (NH, c0 + _PW))
        pancols = []
        for j in range(_PW):
            cj = c0 + j
            colv = jax.lax.slice(Apan, (0, j), (NH, j + 1))
            akk = jax.lax.slice(Apan, (cj, j), (cj + 1, j + 1))
            rinv = jax.lax.rsqrt(akk)
            recips.append(rinv)
            row_ge = jnp.where(rows >= cj, one, zero)
            pancols.append(colv * rinv * row_ge)
            if j < _PW - 1:
                rowv8 = jax.lax.slice(Apan, (cj, 0), (cj + 1, _PW))
                row_gt = jnp.where(rows > cj, one, zero)
                lane_gt = jnp.where(lanes8 > j, one, zero)
                Apan = Apan - (colv * row_gt) * (rowv8 * lane_gt * (rinv * rinv))
        Lpan = jnp.concatenate(pancols, axis=1)
        lpans.append(Lpan)
        if p < _NP - 1:
            A = A - jnp.dot(Lpan, Lpan.T,
                            preferred_element_type=jnp.float32)
    L = jnp.concatenate(lpans, axis=1)
    Lt = L.T
    X = jnp.zeros((NH, NH), jnp.float32)
    for p in range(_NP):
        c0 = p * _PW
        lt_pan = jax.lax.slice(Lt, (0, c0), (NH, c0 + _PW))
        acc8 = jnp.dot(X, lt_pan, preferred_element_type=jnp.float32)
        xcols = []
        for j in range(_PW):
            cj = c0 + j
            acc = jax.lax.slice(acc8, (0, j), (NH, j + 1))
            for i in range(j):
                lji = jax.lax.slice(L, (cj, c0 + i), (cj + 1, c0 + i + 1))
                acc = acc + xcols[i] * lji
            wcol = jax.lax.slice(W, (0, cj), (NH, cj + 1))
            xcols.append((wcol - acc) * recips[cj])
        Xpan = jnp.concatenate(xcols, axis=1)
        sel = (lanes == rows8 + c0).astype(jnp.float32)
        X = X + jnp.dot(Xpan, sel, preferred_element_type=jnp.float32)
    return X


def _logsoftmax(z):
    m = jnp.max(z, axis=1, keepdims=True)
    e = z - m
    return e - jnp.log(jnp.sum(jnp.exp(e), axis=1, keepdims=True))


def _fc_body(x_ref, wt_ref, b_ref, o_ref):
    f0 = jnp.dot(x_ref[0], wt_ref[0],
                 preferred_element_type=jnp.float32) + b_ref[0]
    f1 = jnp.dot(x_ref[1], wt_ref[1],
                 preferred_element_type=jnp.float32) + b_ref[1]
    o_ref[...] = jnp.maximum(jnp.concatenate([f0, f1], axis=1), 0.0)


def _fc_stage(x, fc_Wt, fc_b3):
    return pl.pallas_call(
        _fc_body,
        in_specs=[
            pl.BlockSpec((K, N, NFEAT), lambda: (0, 0, 0)),
            pl.BlockSpec((K, NFEAT, NH), lambda: (0, 0, 0)),
            pl.BlockSpec((K, 1, NH), lambda: (0, 0, 0)),
        ],
        out_specs=pl.BlockSpec((N, K * NH), lambda: (0, 0)),
        out_shape=jax.ShapeDtypeStruct((N, K * NH), jnp.float32),
    )(x, fc_Wt, fc_b3)


_LO = slice(0, NH)
_HI = slice(NH, 2 * NH)


def _mega_body(adj_ref, fcpk_ref, convw_ref, fcowt_ref, fcob_ref, w_ref,
               fin_ref, mean_ref, logs_ref, p1, p2, p3, msc):
    t = pl.program_id(0)
    i = t % NT
    ds = pl.ds(i * BM, BM)
    w = w_ref[0, 0]
    a = jnp.float32(ALPHA)
    na = jnp.float32(1.0 - ALPHA)

    @pl.when(t == 0)
    def _prologue():
        rows = jax.lax.broadcasted_iota(jnp.int32, (NH, 1), 0)
        lanes = jax.lax.broadcasted_iota(jnp.int32, (1, NH), 1)
        eye = (rows == lanes).astype(jnp.float32)
        X0 = _chol_solve(convw_ref[0], rows, lanes, eye)
        msc[0] = T0 * X0 + (1.0 - T0) * eye
        X1 = _chol_solve(convw_ref[1], rows, lanes, eye)
        msc[1] = T1 * X1 + (1.0 - T1) * eye

    @pl.when(t < NT)
    def _l0():
        hi = jnp.dot(adj_ref[0], fcpk_ref[:, _LO],
                     preferred_element_type=jnp.float32)
        support = na * hi + a * fcpk_ref[ds, _LO]
        z = jnp.dot(support, msc[0], preferred_element_type=jnp.float32)
        out = jnp.maximum(jnp.tanh(z), 0.0)
        p1[ds, _LO] = out
        p2[ds, _LO] = w * fcpk_ref[ds, _HI] + (1.0 - w) * out

    @pl.when((t >= NT) & (t < 2 * NT))
    def _l1():
        hi = jnp.dot(adj_ref[0], p1[:, _LO],
                     preferred_element_type=jnp.float32)
        support = na * hi + a * fcpk_ref[ds, _LO]
        z = jnp.dot(support, msc[1], preferred_element_type=jnp.float32)
        p1[ds, _HI] = jnp.maximum(jnp.tanh(z), 0.0)

    @pl.when((t >= 2 * NT) & (t < 3 * NT))
    def _l2():
        hi = jnp.dot(adj_ref[0], p2[:, _LO],
                     preferred_element_type=jnp.float32)
        support = na * hi + a * fcpk_ref[ds, _HI]
        z = jnp.dot(support, msc[0], preferred_element_type=jnp.float32)
        out = jnp.maximum(jnp.tanh(z), 0.0)
        p2[ds, _HI] = out
        p3[ds, _LO] = w * out + (1.0 - w) * p1[ds, _HI]

    @pl.when(t >= 3 * NT)
    def _l3():
        hi = jnp.dot(adj_ref[0], p3[:, _LO],
                     preferred_element_type=jnp.float32)
        support = na * hi + a * fcpk_ref[ds, _HI]
        z = jnp.dot(support, msc[1], preferred_element_type=jnp.float32)
        out11 = jnp.maximum(jnp.tanh(z), 0.0)
        s0 = p1[ds, _LO] + p1[ds, _HI]
        s1 = p2[ds, _HI] + out11
        wt = fcowt_ref[...]
        b = fcob_ref[...]
        lg0 = jnp.dot(s0, wt, preferred_element_type=jnp.float32) + b
        lg1 = jnp.dot(s1, wt, preferred_element_type=jnp.float32) + b
        ls0 = _logsoftmax(lg0)
        ls1 = _logsoftmax(lg1)
        fin_ref[...] = _logsoftmax(lg0 + lg1)
        mean_ref[...] = 0.5 * (ls0 + ls1)
        logs_ref[0] = ls0
        logs_ref[1] = ls1


def kernel(x, adj, conv_W, fc_W, fc_b, fco_W, fco_b, w):
    fc_Wt = jnp.swapaxes(fc_W, 1, 2)
    fc_b3 = fc_b[:, None, :]
    fco_Wt = fco_W.T
    fco_b2 = fco_b[None, :]
    w2d = w.reshape(1, 1)

    fcpk = _fc_stage(x, fc_Wt, fc_b3)

    fin, mean, logs = pl.pallas_call(
        _mega_body,
        grid=(4 * NT,),
        in_specs=[
            pl.BlockSpec((1, BM, N), lambda t: (t // (2 * NT), t % NT, 0)),
            pl.BlockSpec((N, K * NH), lambda t: (0, 0)),
            pl.BlockSpec((NLAYERS, NH, NH), lambda t: (0, 0, 0)),
            pl.BlockSpec((NH, NCLASS), lambda t: (0, 0)),
            pl.BlockSpec((1, NCLASS), lambda t: (0, 0)),
            pl.BlockSpec(memory_space=pltpu.SMEM),
        ],
        out_specs=(
            pl.BlockSpec((BM, NCLASS),
                         lambda t: (jnp.where(t >= 3 * NT, t % NT, 0), 0)),
            pl.BlockSpec((BM, NCLASS),
                         lambda t: (jnp.where(t >= 3 * NT, t % NT, 0), 0)),
            pl.BlockSpec((K, BM, NCLASS),
                         lambda t: (0, jnp.where(t >= 3 * NT, t % NT, 0), 0)),
        ),
        out_shape=(
            jax.ShapeDtypeStruct((N, NCLASS), jnp.float32),
            jax.ShapeDtypeStruct((N, NCLASS), jnp.float32),
            jax.ShapeDtypeStruct((K, N, NCLASS), jnp.float32),
        ),
        scratch_shapes=[
            pltpu.VMEM((N, K * NH), jnp.float32),
            pltpu.VMEM((N, K * NH), jnp.float32),
            pltpu.VMEM((N, NH), jnp.float32),
            pltpu.VMEM((NLAYERS, NH, NH), jnp.float32),
        ],
        compiler_params=pltpu.CompilerParams(
            dimension_semantics=("arbitrary",),
            vmem_limit_bytes=62 * 1024 * 1024),
    )(adj, fcpk, conv_W, fco_Wt, fco_b2, w2d)
    return fin, mean, logs, w


# speedup vs baseline: 1.0311x; 1.0311x over previous
import math

import jax
import jax.numpy as jnp
from jax.experimental import pallas as pl
from jax.experimental.pallas import tpu as pltpu

K = 2
N = 10000
NFEAT = 128
NH = 64
NCLASS = 40
NLAYERS = 2
LAMDA = 0.5
ALPHA = 0.1

BM = 400
NT = N // BM
T0 = math.log(LAMDA / 1.0 + 1.0)
T1 = math.log(LAMDA / 2.0 + 1.0)


def _chol_solve(W, rows, lanes, eye):
    A = jax.lax.dot_general(W, W, (((0,), (0,)), ((), ())),
                            preferred_element_type=jnp.float32)
    A = A + 1e-4 * eye
    one = jnp.float32(1.0)
    zero = jnp.float32(0.0)
    cols = []
    recips = []
    for k in range(NH):
        colv = jax.lax.slice(A, (0, k), (NH, k + 1))
        akk = jax.lax.slice(A, (k, k), (k + 1, k + 1))
        rowv = jax.lax.slice(A, (k, 0), (k + 1, NH))
        rinv = jax.lax.rsqrt(akk)
        recips.append(rinv)
        row_ge = jnp.where(rows >= k, one, zero)
        cols.append(colv * rinv * row_ge)
        if k < NH - 1:
            row_gt = jnp.where(rows > k, one, zero)
            lane_ge = jnp.where(lanes >= k, one, zero)
            A = A - (colv * row_gt) * (rowv * lane_ge * (rinv * rinv))
    L = jnp.concatenate(cols, axis=1)
    Lt = L.T
    X = jnp.zeros((NH, NH), jnp.float32)
    for j in range(NH):
        ltcol = jax.lax.slice(Lt, (0, j), (NH, j + 1))
        acc = jnp.dot(X, ltcol, preferred_element_type=jnp.float32)
        wcol = jax.lax.slice(W, (0, j), (NH, j + 1))
        xcol = (wcol - acc) * recips[j]
        X = X + xcol * jnp.where(lanes == j, one, zero)
    return X


def _logsoftmax(z):
    m = jnp.max(z, axis=1, keepdims=True)
    e = z - m
    return e - jnp.log(jnp.sum(jnp.exp(e), axis=1, keepdims=True))


def _fc_body(x_ref, wt_ref, b_ref, o_ref):
    f0 = jnp.dot(x_ref[0], wt_ref[0],
                 preferred_element_type=jnp.float32) + b_ref[0]
    f1 = jnp.dot(x_ref[1], wt_ref[1],
                 preferred_element_type=jnp.float32) + b_ref[1]
    o_ref[...] = jnp.maximum(jnp.concatenate([f0, f1], axis=1), 0.0)


def _fc_stage(x, fc_Wt, fc_b3):
    return pl.pallas_call(
        _fc_body,
        in_specs=[
            pl.BlockSpec((K, N, NFEAT), lambda: (0, 0, 0)),
            pl.BlockSpec((K, NFEAT, NH), lambda: (0, 0, 0)),
            pl.BlockSpec((K, 1, NH), lambda: (0, 0, 0)),
        ],
        out_specs=pl.BlockSpec((N, K * NH), lambda: (0, 0)),
        out_shape=jax.ShapeDtypeStruct((N, K * NH), jnp.float32),
    )(x, fc_Wt, fc_b3)


_LO = slice(0, NH)
_HI = slice(NH, 2 * NH)


def _mega_body(adj_ref, fcpk_ref, convw_ref, fcowt_ref, fcob_ref, w_ref,
               fin_ref, mean_ref, logs_ref, p1, p2, p3, msc):
    t = pl.program_id(0)
    i = t % NT
    ds = pl.ds(i * BM, BM)
    w = w_ref[0, 0]
    a = jnp.float32(ALPHA)
    na = jnp.float32(1.0 - ALPHA)

    @pl.when(t == 0)
    def _prologue():
        rows = jax.lax.broadcasted_iota(jnp.int32, (NH, 1), 0)
        lanes = jax.lax.broadcasted_iota(jnp.int32, (1, NH), 1)
        eye = (rows == lanes).astype(jnp.float32)
        X0 = _chol_solve(convw_ref[0], rows, lanes, eye)
        msc[0] = T0 * X0 + (1.0 - T0) * eye
        X1 = _chol_solve(convw_ref[1], rows, lanes, eye)
        msc[1] = T1 * X1 + (1.0 - T1) * eye

    @pl.when(t < NT)
    def _l0():
        hi = jnp.dot(adj_ref[0], fcpk_ref[:, _LO],
                     preferred_element_type=jnp.float32)
        support = na * hi + a * fcpk_ref[ds, _LO]
        z = jnp.dot(support, msc[0], preferred_element_type=jnp.float32)
        out = jnp.maximum(jnp.tanh(z), 0.0)
        p1[ds, _LO] = out
        p2[ds, _LO] = w * fcpk_ref[ds, _HI] + (1.0 - w) * out

    @pl.when((t >= NT) & (t < 2 * NT))
    def _l1():
        hi = jnp.dot(adj_ref[0], p1[:, _LO],
                     preferred_element_type=jnp.float32)
        support = na * hi + a * fcpk_ref[ds, _LO]
        z = jnp.dot(support, msc[1], preferred_element_type=jnp.float32)
        p1[ds, _HI] = jnp.maximum(jnp.tanh(z), 0.0)

    @pl.when((t >= 2 * NT) & (t < 3 * NT))
    def _l2():
        hi = jnp.dot(adj_ref[0], p2[:, _LO],
                     preferred_element_type=jnp.float32)
        support = na * hi + a * fcpk_ref[ds, _HI]
        z = jnp.dot(support, msc[0], preferred_element_type=jnp.float32)
        out = jnp.maximum(jnp.tanh(z), 0.0)
        p2[ds, _HI] = out
        p3[ds, _LO] = w * out + (1.0 - w) * p1[ds, _HI]

    @pl.when(t >= 3 * NT)
    def _l3():
        hi = jnp.dot(adj_ref[0], p3[:, _LO],
                     preferred_element_type=jnp.float32)
        support = na * hi + a * fcpk_ref[ds, _HI]
        z = jnp.dot(support, msc[1], preferred_element_type=jnp.float32)
        out11 = jnp.maximum(jnp.tanh(z), 0.0)
        s0 = p1[ds, _LO] + p1[ds, _HI]
        s1 = p2[ds, _HI] + out11
        wt = fcowt_ref[...]
        b = fcob_ref[...]
        lg0 = jnp.dot(s0, wt, preferred_element_type=jnp.float32) + b
        lg1 = jnp.dot(s1, wt, preferred_element_type=jnp.float32) + b
        ls0 = _logsoftmax(lg0)
        ls1 = _logsoftmax(lg1)
        fin_ref[...] = _logsoftmax(lg0 + lg1)
        mean_ref[...] = 0.5 * (ls0 + ls1)
        logs_ref[0] = ls0
        logs_ref[1] = ls1


def kernel(x, adj, conv_W, fc_W, fc_b, fco_W, fco_b, w):
    fc_Wt = jnp.swapaxes(fc_W, 1, 2)
    fc_b3 = fc_b[:, None, :]
    fco_Wt = fco_W.T
    fco_b2 = fco_b[None, :]
    w2d = w.reshape(1, 1)

    fcpk = _fc_stage(x, fc_Wt, fc_b3)

    fin, mean, logs = pl.pallas_call(
        _mega_body,
        grid=(4 * NT,),
        in_specs=[
            pl.BlockSpec((1, BM, N), lambda t: (t // (2 * NT), t % NT, 0)),
            pl.BlockSpec((N, K * NH), lambda t: (0, 0)),
            pl.BlockSpec((NLAYERS, NH, NH), lambda t: (0, 0, 0)),
            pl.BlockSpec((NH, NCLASS), lambda t: (0, 0)),
            pl.BlockSpec((1, NCLASS), lambda t: (0, 0)),
            pl.BlockSpec(memory_space=pltpu.SMEM),
        ],
        out_specs=(
            pl.BlockSpec((BM, NCLASS),
                         lambda t: (jnp.where(t >= 3 * NT, t % NT, 0), 0)),
            pl.BlockSpec((BM, NCLASS),
                         lambda t: (jnp.where(t >= 3 * NT, t % NT, 0), 0)),
            pl.BlockSpec((K, BM, NCLASS),
                         lambda t: (0, jnp.where(t >= 3 * NT, t % NT, 0), 0)),
        ),
        out_shape=(
            jax.ShapeDtypeStruct((N, NCLASS), jnp.float32),
            jax.ShapeDtypeStruct((N, NCLASS), jnp.float32),
            jax.ShapeDtypeStruct((K, N, NCLASS), jnp.float32),
        ),
        scratch_shapes=[
            pltpu.VMEM((N, K * NH), jnp.float32),
            pltpu.VMEM((N, K * NH), jnp.float32),
            pltpu.VMEM((N, NH), jnp.float32),
            pltpu.VMEM((NLAYERS, NH, NH), jnp.float32),
        ],
        compiler_params=pltpu.CompilerParams(
            dimension_semantics=("arbitrary",),
            vmem_limit_bytes=62 * 1024 * 1024),
    )(adj, fcpk, conv_W, fco_Wt, fco_b2, w2d)
    return fin, mean, logs, w
